# Initial kernel scaffold; baseline (speedup 1.0000x reference)
import functools

import jax
import jax.numpy as jnp
import numpy as np
from jax import lax
from jax.experimental import pallas as pl
from jax.experimental.pallas import tpu as pltpu

N_NODES = 100000
N_EDGES = 1600000
N_SCALAR = 16
N_VEC = 8
EMBED_DIM = 8
N_BASIS = 8
RAD_H = 16
RAD_OUT = 32
FEAT_DIM = N_SCALAR + N_VEC + EMBED_DIM
LAYERS = 2
MAX_RADIUS = 5.0
INV_SQRT_DEG = 1.0 / np.sqrt(16.0)

EDGE_BLOCK = 8000


def _semi_unitary(M):
    I = jnp.eye(M.shape[-2], dtype=M.dtype)
    for _ in range(10):
        M = M - 0.5 * (M @ M.T - I) @ M
    return M


def _expand_rep3(n):
    R = np.zeros((n, 3 * n), dtype=np.float32)
    for c in range(n):
        for d in range(3):
            R[c, 3 * c + d] = 1.0
    return jnp.asarray(R)


def _edge_kernel(ev_ref, g_ref, wcomb_ref, br1_ref, wr2_ref, br2_ref, wout_ref,
                 msg_ref):
    ev = ev_ref[...]
    g = g_ref[...]
    elen = jnp.sqrt(jnp.sum(ev * ev, axis=1, keepdims=True) + 1e-12)
    edir = ev / elen
    values = jnp.linspace(0.0, MAX_RADIUS, N_BASIS).astype(jnp.float32)
    step = MAX_RADIUS / (N_BASIS - 1)
    diff = (elen - values[None, :]) / step
    basis = jnp.exp(-diff * diff) * 0.5
    edir_t = jnp.concatenate([edir] * N_VEC, axis=1)
    vflat = g[:, N_SCALAR:N_SCALAR + 3 * N_VEC]
    xin = jnp.concatenate([g[:, :N_SCALAR], vflat * edir_t,
                           g[:, N_SCALAR + 3 * N_VEC:], basis], axis=1)
    hr = jnp.dot(xin, wcomb_ref[...], preferred_element_type=jnp.float32)
    h_pre = hr[:, :RAD_OUT]
    r_pre = hr[:, RAD_OUT:] + br1_ref[...]
    radial = jnp.dot(jax.nn.silu(r_pre), wr2_ref[...],
                     preferred_element_type=jnp.float32) + br2_ref[...]
    h = jax.nn.silu(h_pre) * radial
    o = jnp.dot(h, wout_ref[...], preferred_element_type=jnp.float32)
    msg_v = o[:, 24:48] * edir_t + o[:, 48:72] * vflat
    msg_ref[...] = jnp.concatenate([o[:, :24], msg_v], axis=1)


def _edge_stage(ev, g, wcomb, br1, wr2, br2, wout):
    nblk = N_EDGES // EDGE_BLOCK
    return pl.pallas_call(
        _edge_kernel,
        grid=(nblk,),
        in_specs=[
            pl.BlockSpec((EDGE_BLOCK, 3), lambda i: (i, 0)),
            pl.BlockSpec((EDGE_BLOCK, 48), lambda i: (i, 0)),
            pl.BlockSpec((56, 48), lambda i: (0, 0)),
            pl.BlockSpec((1, RAD_H), lambda i: (0, 0)),
            pl.BlockSpec((RAD_H, RAD_OUT), lambda i: (0, 0)),
            pl.BlockSpec((1, RAD_OUT), lambda i: (0, 0)),
            pl.BlockSpec((RAD_OUT, 72), lambda i: (0, 0)),
        ],
        out_specs=pl.BlockSpec((EDGE_BLOCK, 48), lambda i: (i, 0)),
        out_shape=jax.ShapeDtypeStruct((N_EDGES, 48), jnp.float32),
    )(ev, g, wcomb, br1, wr2, br2, wout)


def kernel(x, batch, node_attr, edge_index, edge_vec, projection_matrix, embedding,
           W_r1, b_r1, W_r2, b_r2, W_e, W_s, W_v1, W_v2, SI_s, SI_v):
    src = edge_index[0]
    dst = edge_index[1]
    Mu = _semi_unitary(projection_matrix)
    R = _expand_rep3(N_VEC)
    U = jnp.zeros((6, 3 * N_VEC), jnp.float32)
    for k in range(2):
        for d in range(3):
            U = U.at[3 * k + d, d::3].set(Mu[k, :])
    vflat = x @ U
    s = jnp.zeros((N_NODES, N_SCALAR), jnp.float32)
    z = jnp.take(embedding, node_attr, axis=0)
    P = jnp.zeros((3 * N_VEC, 6), jnp.float32)
    for k in range(2):
        for d in range(3):
            P = P.at[d::3, 3 * k + d].set(Mu[k, :])
    for l in range(LAYERS):
        wcomb = jnp.zeros((56, 48), jnp.float32)
        wcomb = wcomb.at[:N_SCALAR, :RAD_OUT].set(W_e[l, :N_SCALAR, :])
        wcomb = wcomb.at[N_SCALAR:N_SCALAR + 24, :RAD_OUT].set(
            R.T @ W_e[l, N_SCALAR:N_SCALAR + N_VEC, :])
        wcomb = wcomb.at[N_SCALAR + 24:N_SCALAR + 32, :RAD_OUT].set(
            W_e[l, N_SCALAR + N_VEC:, :])
        wcomb = wcomb.at[N_SCALAR + 32:, RAD_OUT:].set(W_r1[l])
        wout = jnp.concatenate([W_s[l], W_v1[l] @ R, W_v2[l] @ R], axis=1)
        table = jnp.concatenate([s, vflat, z], axis=1)
        g = jnp.take(table, src, axis=0)
        msg = _edge_stage(edge_vec, g, wcomb, b_r1[l][None, :], W_r2[l],
                          b_r2[l][None, :], wout)
        agg = jnp.zeros((N_NODES, 48), jnp.float32).at[dst].add(msg) * INV_SQRT_DEG
        scal = s + agg[:, :N_SCALAR]
        gates = agg[:, N_SCALAR:24]
        s = jax.nn.silu(scal @ SI_s[l])
        SIvE = jnp.zeros((24, 24), jnp.float32)
        for d in range(3):
            SIvE = SIvE.at[d::3, d::3].set(SI_v[l])
        w = vflat + agg[:, 24:]
        vflat = (jax.nn.sigmoid(gates) @ R) * (w @ SIvE)
    return vflat @ P


# speedup vs baseline: 16.1234x; 16.1234x over previous
import functools

import jax
import jax.numpy as jnp
import numpy as np
from jax import lax
from jax.experimental import pallas as pl
from jax.experimental.pallas import tpu as pltpu

N_NODES = 100000
N_EDGES = 1600000
N_SCALAR = 16
N_VEC = 8
EMBED_DIM = 8
N_BASIS = 8
RAD_H = 16
RAD_OUT = 32
FEAT_DIM = N_SCALAR + N_VEC + EMBED_DIM
LAYERS = 2
MAX_RADIUS = 5.0
INV_SQRT_DEG = 1.0 / np.sqrt(16.0)

EDGE_BLOCK = 8000


def _semi_unitary(M):
    I = jnp.eye(M.shape[-2], dtype=M.dtype)
    for _ in range(10):
        M = M - 0.5 * (M @ M.T - I) @ M
    return M


def _expand_rep3(n):
    R = np.zeros((n, 3 * n), dtype=np.float32)
    for c in range(n):
        for d in range(3):
            R[c, 3 * c + d] = 1.0
    return jnp.asarray(R)


def _edge_kernel(ev_ref, g_ref, wcomb_ref, br1_ref, wr2_ref, br2_ref, wout_ref,
                 msg_ref):
    ev = ev_ref[...]
    g = g_ref[...]
    elen = jnp.sqrt(jnp.sum(ev * ev, axis=1, keepdims=True) + 1e-12)
    edir = ev / elen
    step = MAX_RADIUS / (N_BASIS - 1)
    values = lax.broadcasted_iota(jnp.int32, (1, N_BASIS), 1).astype(jnp.float32) * step
    diff = (elen - values) / step
    basis = jnp.exp(-diff * diff) * 0.5
    edir_t = jnp.concatenate([edir] * N_VEC, axis=1)
    vflat = g[:, N_SCALAR:N_SCALAR + 3 * N_VEC]
    xin = jnp.concatenate([g[:, :N_SCALAR], vflat * edir_t,
                           g[:, N_SCALAR + 3 * N_VEC:], basis], axis=1)
    hr = jnp.dot(xin, wcomb_ref[...], preferred_element_type=jnp.float32)
    h_pre = hr[:, :RAD_OUT]
    r_pre = hr[:, RAD_OUT:] + br1_ref[...]
    radial = jnp.dot(jax.nn.silu(r_pre), wr2_ref[...],
                     preferred_element_type=jnp.float32) + br2_ref[...]
    h = jax.nn.silu(h_pre) * radial
    o = jnp.dot(h, wout_ref[...], preferred_element_type=jnp.float32)
    msg_v = o[:, 24:48] * edir_t + o[:, 48:72] * vflat
    msg_ref[...] = jnp.concatenate([o[:, :24], msg_v], axis=1)


def _edge_stage(ev, g, wcomb, br1, wr2, br2, wout):
    nblk = N_EDGES // EDGE_BLOCK
    return pl.pallas_call(
        _edge_kernel,
        grid=(nblk,),
        in_specs=[
            pl.BlockSpec((EDGE_BLOCK, 3), lambda i: (i, 0)),
            pl.BlockSpec((EDGE_BLOCK, 48), lambda i: (i, 0)),
            pl.BlockSpec((56, 48), lambda i: (0, 0)),
            pl.BlockSpec((1, RAD_H), lambda i: (0, 0)),
            pl.BlockSpec((RAD_H, RAD_OUT), lambda i: (0, 0)),
            pl.BlockSpec((1, RAD_OUT), lambda i: (0, 0)),
            pl.BlockSpec((RAD_OUT, 72), lambda i: (0, 0)),
        ],
        out_specs=pl.BlockSpec((EDGE_BLOCK, 48), lambda i: (i, 0)),
        out_shape=jax.ShapeDtypeStruct((N_EDGES, 48), jnp.float32),
    )(ev, g, wcomb, br1, wr2, br2, wout)


def kernel(x, batch, node_attr, edge_index, edge_vec, projection_matrix, embedding,
           W_r1, b_r1, W_r2, b_r2, W_e, W_s, W_v1, W_v2, SI_s, SI_v):
    src = edge_index[0]
    dst = edge_index[1]
    Mu = _semi_unitary(projection_matrix)
    R = _expand_rep3(N_VEC)
    U = jnp.zeros((6, 3 * N_VEC), jnp.float32)
    for k in range(2):
        for d in range(3):
            U = U.at[3 * k + d, d::3].set(Mu[k, :])
    vflat = x @ U
    s = jnp.zeros((N_NODES, N_SCALAR), jnp.float32)
    z = jnp.take(embedding, node_attr, axis=0)
    P = jnp.zeros((3 * N_VEC, 6), jnp.float32)
    for k in range(2):
        for d in range(3):
            P = P.at[d::3, 3 * k + d].set(Mu[k, :])
    for l in range(LAYERS):
        wcomb = jnp.zeros((56, 48), jnp.float32)
        wcomb = wcomb.at[:N_SCALAR, :RAD_OUT].set(W_e[l, :N_SCALAR, :])
        wcomb = wcomb.at[N_SCALAR:N_SCALAR + 24, :RAD_OUT].set(
            R.T @ W_e[l, N_SCALAR:N_SCALAR + N_VEC, :])
        wcomb = wcomb.at[N_SCALAR + 24:N_SCALAR + 32, :RAD_OUT].set(
            W_e[l, N_SCALAR + N_VEC:, :])
        wcomb = wcomb.at[N_SCALAR + 32:, RAD_OUT:].set(W_r1[l])
        wout = jnp.concatenate([W_s[l], W_v1[l] @ R, W_v2[l] @ R], axis=1)
        table = jnp.concatenate([s, vflat, z], axis=1)
        g = jnp.take(table, src, axis=0)
        msg = _edge_stage(edge_vec, g, wcomb, b_r1[l][None, :], W_r2[l],
                          b_r2[l][None, :], wout)
        agg = jnp.zeros((N_NODES, 48), jnp.float32).at[dst].add(msg) * INV_SQRT_DEG
        scal = s + agg[:, :N_SCALAR]
        gates = agg[:, N_SCALAR:24]
        s = jax.nn.silu(scal @ SI_s[l])
        SIvE = jnp.zeros((24, 24), jnp.float32)
        for d in range(3):
            SIvE = SIvE.at[d::3, d::3].set(SI_v[l])
        w = vflat + agg[:, 24:]
        vflat = (jax.nn.sigmoid(gates) @ R) * (w @ SIvE)
    return vflat @ P


# speedup vs baseline: 36.7623x; 2.2801x over previous
import functools

import jax
import jax.numpy as jnp
import numpy as np
from jax import lax
from jax.experimental import pallas as pl
from jax.experimental.pallas import tpu as pltpu
from jax.experimental.pallas import tpu_sc as plsc

N_NODES = 100000
N_EDGES = 1600000
N_SCALAR = 16
N_VEC = 8
EMBED_DIM = 8
N_BASIS = 8
RAD_H = 16
RAD_OUT = 32
FEAT_DIM = N_SCALAR + N_VEC + EMBED_DIM
LAYERS = 2
MAX_RADIUS = 5.0
INV_SQRT_DEG = 1.0 / np.sqrt(16.0)

EDGE_BLOCK = 2000


def _semi_unitary(M):
    I = jnp.eye(M.shape[-2], dtype=M.dtype)
    for _ in range(10):
        M = M - 0.5 * (M @ M.T - I) @ M
    return M


def _expand_rep3(n):
    R = np.zeros((n, 3 * n), dtype=np.float32)
    for c in range(n):
        for d in range(3):
            R[c, 3 * c + d] = 1.0
    return jnp.asarray(R)


GATHER_WIN = 128


def _sc_mesh():
    return plsc.VectorSubcoreMesh(core_axis_name="c", subcore_axis_name="s")


GBATCH = 5


def _sc_gather(table, idx):
    n_step = N_EDGES // (GATHER_WIN * GBATCH)
    idx2 = idx.reshape(n_step, GBATCH, GATHER_WIN)

    @functools.partial(
        pl.kernel,
        out_type=jax.ShapeDtypeStruct((N_EDGES, 48), jnp.float32),
        mesh=_sc_mesh(),
        scratch_types=[pltpu.SemaphoreType.DMA],
        compiler_params=pltpu.CompilerParams(use_tc_tiling_on_sc=False),
    )
    def k(table_hbm, idx_hbm, out_hbm, sem):
        def body(i_vmem, o_vmem):
            cps = [
                pltpu.async_copy(
                    table_hbm.at[i_vmem.at[0, j]],
                    o_vmem.at[pl.ds(j * GATHER_WIN, GATHER_WIN)], sem)
                for j in range(GBATCH)
            ]
            for cp in cps:
                cp.wait()

        pltpu.emit_pipeline(
            body,
            grid=(n_step,),
            in_specs=[pl.BlockSpec((1, GBATCH, GATHER_WIN),
                                   index_map=lambda i: (i, 0, 0))],
            out_specs=[pl.BlockSpec((GATHER_WIN * GBATCH, 48),
                                    index_map=lambda i: (i, 0))],
            core_axis_name=("c", "s"),
            dimension_semantics=(pltpu.PARALLEL,),
        )(idx_hbm, out_hbm)

    return k(table, idx2)


SUB_ROWS = N_NODES // 16


def _sc_scatter(msg, idx2, zeros, grp):
    n_step = N_EDGES // (GATHER_WIN * GBATCH)
    idx3 = idx2.reshape(n_step, GBATCH, GATHER_WIN)

    @functools.partial(
        pl.kernel,
        out_type=jax.ShapeDtypeStruct((2, N_NODES, 16), jnp.float32),
        mesh=_sc_mesh(),
        scratch_types=[pltpu.VMEM_SHARED((N_NODES, 16), jnp.float32),
                       pltpu.SemaphoreType.DMA],
        compiler_params=pltpu.CompilerParams(use_tc_tiling_on_sc=False),
    )
    def k(msg_hbm, idx_hbm, zeros_hbm, out_hbm, acc, sem):
        cid = lax.axis_index("c")
        sid = lax.axis_index("s")
        sl = pl.ds(sid * SUB_ROWS, SUB_ROWS)
        pltpu.sync_copy(zeros_hbm.at[sl], acc.at[sl])
        plsc.subcore_barrier()

        def body(x_vmem, i_vmem):
            cps = [
                pltpu.async_copy(
                    x_vmem.at[pl.ds(j * GATHER_WIN, GATHER_WIN)],
                    acc.at[i_vmem.at[0, j]], sem, add=True)
                for j in range(GBATCH)
            ]
            for cp in cps:
                cp.wait()

        pltpu.emit_pipeline(
            body,
            grid=(n_step,),
            in_specs=[
                pl.BlockSpec((GATHER_WIN * GBATCH, 16),
                             index_map=lambda i: (i, grp)),
                pl.BlockSpec((1, GBATCH, GATHER_WIN),
                             index_map=lambda i: (i, 0, 0)),
            ],
            out_specs=[],
            core_axis_name=("c", "s"),
            dimension_semantics=(pltpu.PARALLEL,),
        )(msg_hbm, idx_hbm)
        plsc.subcore_barrier()
        pltpu.sync_copy(acc.at[sl], out_hbm.at[cid].at[sl])

    return k(msg, idx3, zeros)


def _edge_kernel(ev_ref, g_ref, wcomb_ref, br1_ref, wr2_ref, br2_ref, wout_ref,
                 msg_ref):
    ev = ev_ref[...]
    g = g_ref[...]
    elen = jnp.sqrt(jnp.sum(ev * ev, axis=1, keepdims=True) + 1e-12)
    edir = ev / elen
    step = MAX_RADIUS / (N_BASIS - 1)
    values = lax.broadcasted_iota(jnp.int32, (1, N_BASIS), 1).astype(jnp.float32) * step
    diff = (elen - values) / step
    basis = jnp.exp(-diff * diff) * 0.5
    edir_t = jnp.concatenate([edir] * N_VEC, axis=1)
    vflat = g[:, N_SCALAR:N_SCALAR + 3 * N_VEC]
    xin = jnp.concatenate([g[:, :N_SCALAR], vflat * edir_t,
                           g[:, N_SCALAR + 3 * N_VEC:], basis], axis=1)
    hr = jnp.dot(xin, wcomb_ref[...], preferred_element_type=jnp.float32)
    h_pre = hr[:, :RAD_OUT]
    r_pre = hr[:, RAD_OUT:] + br1_ref[...]
    radial = jnp.dot(jax.nn.silu(r_pre), wr2_ref[...],
                     preferred_element_type=jnp.float32) + br2_ref[...]
    h = jax.nn.silu(h_pre) * radial
    o = jnp.dot(h, wout_ref[...], preferred_element_type=jnp.float32)
    msg_v = o[:, 24:48] * edir_t + o[:, 48:72] * vflat
    msg_ref[...] = jnp.concatenate([o[:, :24], msg_v], axis=1)


def _edge_stage(ev, g, wcomb, br1, wr2, br2, wout):
    nblk = N_EDGES // EDGE_BLOCK
    return pl.pallas_call(
        _edge_kernel,
        grid=(nblk,),
        in_specs=[
            pl.BlockSpec((EDGE_BLOCK, 3), lambda i: (i, 0)),
            pl.BlockSpec((EDGE_BLOCK, 48), lambda i: (i, 0)),
            pl.BlockSpec((56, 48), lambda i: (0, 0)),
            pl.BlockSpec((1, RAD_H), lambda i: (0, 0)),
            pl.BlockSpec((RAD_H, RAD_OUT), lambda i: (0, 0)),
            pl.BlockSpec((1, RAD_OUT), lambda i: (0, 0)),
            pl.BlockSpec((RAD_OUT, 72), lambda i: (0, 0)),
        ],
        out_specs=pl.BlockSpec((EDGE_BLOCK, 48), lambda i: (i, 0)),
        out_shape=jax.ShapeDtypeStruct((N_EDGES, 48), jnp.float32),
    )(ev, g, wcomb, br1, wr2, br2, wout)


NODE_BLOCK = 2000
NMAX_ATOMS = 20


def _prep_kernel(x_ref, attr_ref, u_ref, emb_ref, tab_ref):
    xb = x_ref[...]
    vf = jnp.dot(xb, u_ref[...], preferred_element_type=jnp.float32)
    oh = (attr_ref[...] == lax.broadcasted_iota(
        jnp.int32, (NODE_BLOCK, NMAX_ATOMS), 1)).astype(jnp.float32)
    zb = jnp.dot(oh, emb_ref[...], preferred_element_type=jnp.float32)
    tab_ref[...] = jnp.concatenate(
        [jnp.zeros((NODE_BLOCK, N_SCALAR), jnp.float32), vf, zb], axis=1)


def _prep_nodes(x, attr2, U, embedding):
    nblk = N_NODES // NODE_BLOCK
    return pl.pallas_call(
        _prep_kernel,
        grid=(nblk,),
        in_specs=[
            pl.BlockSpec((NODE_BLOCK, 6), lambda i: (i, 0)),
            pl.BlockSpec((NODE_BLOCK, 1), lambda i: (i, 0)),
            pl.BlockSpec((6, 24), lambda i: (0, 0)),
            pl.BlockSpec((NMAX_ATOMS, EMBED_DIM), lambda i: (0, 0)),
        ],
        out_specs=pl.BlockSpec((NODE_BLOCK, 48), lambda i: (i, 0)),
        out_shape=jax.ShapeDtypeStruct((N_NODES, 48), jnp.float32),
    )(x, attr2, U, embedding)


def _node_kernel(tab_ref, p0_ref, p1_ref, p2_ref, sis_ref, sive_ref, r_ref,
                 p_ref, out_ref, *, final):
    tab = tab_ref[...]
    agg = jnp.concatenate([
        p0_ref[0] + p0_ref[1],
        p1_ref[0] + p1_ref[1],
        p2_ref[0] + p2_ref[1],
    ], axis=1) * INV_SQRT_DEG
    s_old = tab[:, :N_SCALAR]
    vflat_old = tab[:, N_SCALAR:N_SCALAR + 24]
    scal = s_old + agg[:, :N_SCALAR]
    gates = agg[:, N_SCALAR:24]
    s_new = jax.nn.silu(jnp.dot(scal, sis_ref[...],
                                preferred_element_type=jnp.float32))
    w = vflat_old + agg[:, 24:]
    vflat_new = (jnp.dot(jax.nn.sigmoid(gates), r_ref[...],
                         preferred_element_type=jnp.float32) *
                 jnp.dot(w, sive_ref[...], preferred_element_type=jnp.float32))
    if final:
        out_ref[...] = jnp.dot(vflat_new, p_ref[...],
                               preferred_element_type=jnp.float32)
    else:
        out_ref[...] = jnp.concatenate(
            [s_new, vflat_new, tab[:, N_SCALAR + 24:]], axis=1)


def _node_update(table, p0, p1, p2, SIs, SIvE, R, P, final):
    nblk = N_NODES // NODE_BLOCK
    out_w = 6 if final else 48
    return pl.pallas_call(
        functools.partial(_node_kernel, final=final),
        grid=(nblk,),
        in_specs=[
            pl.BlockSpec((NODE_BLOCK, 48), lambda i: (i, 0)),
            pl.BlockSpec((2, NODE_BLOCK, 16), lambda i: (0, i, 0)),
            pl.BlockSpec((2, NODE_BLOCK, 16), lambda i: (0, i, 0)),
            pl.BlockSpec((2, NODE_BLOCK, 16), lambda i: (0, i, 0)),
            pl.BlockSpec((N_SCALAR, N_SCALAR), lambda i: (0, 0)),
            pl.BlockSpec((24, 24), lambda i: (0, 0)),
            pl.BlockSpec((N_VEC, 24), lambda i: (0, 0)),
            pl.BlockSpec((24, 6), lambda i: (0, 0)),
        ],
        out_specs=pl.BlockSpec((NODE_BLOCK, out_w), lambda i: (i, 0)),
        out_shape=jax.ShapeDtypeStruct((N_NODES, out_w), jnp.float32),
    )(table, p0, p1, p2, SIs, SIvE, R, P)


def kernel(x, batch, node_attr, edge_index, edge_vec, projection_matrix, embedding,
           W_r1, b_r1, W_r2, b_r2, W_e, W_s, W_v1, W_v2, SI_s, SI_v):
    src = edge_index[0]
    dst = edge_index[1]
    Mu = _semi_unitary(projection_matrix)
    R = _expand_rep3(N_VEC)
    U = jnp.zeros((6, 3 * N_VEC), jnp.float32)
    for k in range(2):
        for d in range(3):
            U = U.at[3 * k + d, d::3].set(Mu[k, :])
    P = jnp.zeros((3 * N_VEC, 6), jnp.float32)
    for k in range(2):
        for d in range(3):
            P = P.at[d::3, 3 * k + d].set(Mu[k, :])
    table = _prep_nodes(x, node_attr.reshape(N_NODES, 1).astype(jnp.int32),
                        U, embedding)
    src2 = src.reshape(1, N_EDGES)
    dst2 = dst.reshape(1, N_EDGES)
    zeros16 = jnp.zeros((N_NODES, 16), jnp.float32)
    out = None
    for l in range(LAYERS):
        wcomb = jnp.zeros((56, 48), jnp.float32)
        wcomb = wcomb.at[:N_SCALAR, :RAD_OUT].set(W_e[l, :N_SCALAR, :])
        wcomb = wcomb.at[N_SCALAR:N_SCALAR + 24, :RAD_OUT].set(
            R.T @ W_e[l, N_SCALAR:N_SCALAR + N_VEC, :])
        wcomb = wcomb.at[N_SCALAR + 24:N_SCALAR + 32, :RAD_OUT].set(
            W_e[l, N_SCALAR + N_VEC:, :])
        wcomb = wcomb.at[N_SCALAR + 32:, RAD_OUT:].set(W_r1[l])
        wout = jnp.concatenate([W_s[l], W_v1[l] @ R, W_v2[l] @ R], axis=1)
        g = _sc_gather(table, src2)
        msg = _edge_stage(edge_vec, g, wcomb, b_r1[l][None, :], W_r2[l],
                          b_r2[l][None, :], wout)
        p0 = _sc_scatter(msg, dst2, zeros16, 0)
        p1 = _sc_scatter(msg, dst2, zeros16, 1)
        p2 = _sc_scatter(msg, dst2, zeros16, 2)
        SIvE = jnp.zeros((24, 24), jnp.float32)
        for d in range(3):
            SIvE = SIvE.at[d::3, d::3].set(SI_v[l])
        final = l == LAYERS - 1
        nxt = _node_update(table, p0, p1, p2, SI_s[l], SIvE, R, P, final)
        if final:
            out = nxt
        else:
            table = nxt
    return out


# speedup vs baseline: 104.8704x; 2.8527x over previous
import functools

import jax
import jax.numpy as jnp
import numpy as np
from jax import lax
from jax.experimental import pallas as pl
from jax.experimental.pallas import tpu as pltpu
from jax.experimental.pallas import tpu_sc as plsc

N_NODES = 100000
N_EDGES = 1600000
N_SCALAR = 16
N_VEC = 8
EMBED_DIM = 8
N_BASIS = 8
RAD_H = 16
RAD_OUT = 32
FEAT_DIM = N_SCALAR + N_VEC + EMBED_DIM
LAYERS = 2
MAX_RADIUS = 5.0
INV_SQRT_DEG = 1.0 / np.sqrt(16.0)

EDGE_BLOCK = 2000


def _semi_unitary(M):
    I = jnp.eye(M.shape[-2], dtype=M.dtype)
    for _ in range(10):
        M = M - 0.5 * (M @ M.T - I) @ M
    return M


def _expand_rep3(n):
    R = np.zeros((n, 3 * n), dtype=np.float32)
    for c in range(n):
        for d in range(3):
            R[c, 3 * c + d] = 1.0
    return jnp.asarray(R)


GATHER_WIN = 128


def _sc_mesh():
    return plsc.VectorSubcoreMesh(core_axis_name="c", subcore_axis_name="s")


GBATCH = 5


def _sc_gather(table, idx):
    n_step = N_EDGES // (GATHER_WIN * GBATCH)
    idx2 = idx.reshape(1, N_EDGES)

    @functools.partial(
        pl.kernel,
        out_type=jax.ShapeDtypeStruct((N_EDGES, 48), jnp.float32),
        mesh=_sc_mesh(),
        scratch_types=[pltpu.SemaphoreType.DMA],
        compiler_params=pltpu.CompilerParams(use_tc_tiling_on_sc=False),
    )
    def k(table_hbm, idx_hbm, out_hbm, sem):
        def body(*refs):
            idx_refs, o_vmem = refs[:GBATCH], refs[GBATCH]
            cps = [
                pltpu.async_copy(
                    table_hbm.at[idx_refs[j].at[0]],
                    o_vmem.at[pl.ds(j * GATHER_WIN, GATHER_WIN)], sem)
                for j in range(GBATCH)
            ]
            for cp in cps:
                cp.wait()

        pltpu.emit_pipeline(
            body,
            grid=(n_step,),
            in_specs=[pl.BlockSpec((1, GATHER_WIN),
                                   index_map=functools.partial(
                                       lambda j, i: (0, GBATCH * i + j), j))
                      for j in range(GBATCH)],
            out_specs=[pl.BlockSpec((GATHER_WIN * GBATCH, 48),
                                    index_map=lambda i: (i, 0))],
            core_axis_name=("c", "s"),
            dimension_semantics=(pltpu.PARALLEL,),
        )(*([idx_hbm] * GBATCH), out_hbm)

    return k(table, idx2)


SUB_ROWS = N_NODES // 16


def _sc_scatter(msg, idx2, zeros, grp):
    n_win = N_EDGES // GATHER_WIN

    @functools.partial(
        pl.kernel,
        out_type=jax.ShapeDtypeStruct((2, N_NODES, 16), jnp.float32),
        mesh=_sc_mesh(),
        scratch_types=[pltpu.VMEM_SHARED((N_NODES, 16), jnp.float32)],
        compiler_params=pltpu.CompilerParams(use_tc_tiling_on_sc=False),
    )
    def k(msg_hbm, idx_hbm, zeros_hbm, out_hbm, acc):
        cid = lax.axis_index("c")
        sid = lax.axis_index("s")
        sl = pl.ds(sid * SUB_ROWS, SUB_ROWS)
        pltpu.sync_copy(zeros_hbm.at[sl], acc.at[sl])
        plsc.subcore_barrier()

        def body(x_vmem, i_vmem):
            pltpu.sync_copy(x_vmem, acc.at[i_vmem.at[0]], add=True)

        pltpu.emit_pipeline(
            body,
            grid=(n_win,),
            in_specs=[
                pl.BlockSpec((GATHER_WIN, 16), index_map=lambda i: (i, grp)),
                pl.BlockSpec((1, GATHER_WIN), index_map=lambda i: (0, i)),
            ],
            out_specs=[],
            core_axis_name=("c", "s"),
            dimension_semantics=(pltpu.PARALLEL,),
        )(msg_hbm, idx_hbm)
        plsc.subcore_barrier()
        pltpu.sync_copy(acc.at[sl], out_hbm.at[cid].at[sl])

    return k(msg, idx2, zeros)


def _edge_kernel(ev_ref, g_ref, wcomb_ref, br1_ref, wr2_ref, br2_ref, wout_ref,
                 msg_ref):
    ev = ev_ref[...]
    g = g_ref[...]
    elen = jnp.sqrt(jnp.sum(ev * ev, axis=1, keepdims=True) + 1e-12)
    edir = ev / elen
    step = MAX_RADIUS / (N_BASIS - 1)
    values = lax.broadcasted_iota(jnp.int32, (1, N_BASIS), 1).astype(jnp.float32) * step
    diff = (elen - values) / step
    basis = jnp.exp(-diff * diff) * 0.5
    edir_t = jnp.concatenate([edir] * N_VEC, axis=1)
    vflat = g[:, N_SCALAR:N_SCALAR + 3 * N_VEC]
    xin = jnp.concatenate([g[:, :N_SCALAR], vflat * edir_t,
                           g[:, N_SCALAR + 3 * N_VEC:], basis], axis=1)
    hr = jnp.dot(xin, wcomb_ref[...], preferred_element_type=jnp.float32)
    h_pre = hr[:, :RAD_OUT]
    r_pre = hr[:, RAD_OUT:] + br1_ref[...]
    radial = jnp.dot(jax.nn.silu(r_pre), wr2_ref[...],
                     preferred_element_type=jnp.float32) + br2_ref[...]
    h = jax.nn.silu(h_pre) * radial
    o = jnp.dot(h, wout_ref[...], preferred_element_type=jnp.float32)
    msg_v = o[:, 24:48] * edir_t + o[:, 48:72] * vflat
    msg_ref[...] = jnp.concatenate([o[:, :24], msg_v], axis=1)


def _edge_stage(ev, g, wcomb, br1, wr2, br2, wout):
    nblk = N_EDGES // EDGE_BLOCK
    return pl.pallas_call(
        _edge_kernel,
        grid=(nblk,),
        in_specs=[
            pl.BlockSpec((EDGE_BLOCK, 3), lambda i: (i, 0)),
            pl.BlockSpec((EDGE_BLOCK, 48), lambda i: (i, 0)),
            pl.BlockSpec((56, 48), lambda i: (0, 0)),
            pl.BlockSpec((1, RAD_H), lambda i: (0, 0)),
            pl.BlockSpec((RAD_H, RAD_OUT), lambda i: (0, 0)),
            pl.BlockSpec((1, RAD_OUT), lambda i: (0, 0)),
            pl.BlockSpec((RAD_OUT, 72), lambda i: (0, 0)),
        ],
        out_specs=pl.BlockSpec((EDGE_BLOCK, 48), lambda i: (i, 0)),
        out_shape=jax.ShapeDtypeStruct((N_EDGES, 48), jnp.float32),
    )(ev, g, wcomb, br1, wr2, br2, wout)


NODE_BLOCK = 2000
NMAX_ATOMS = 20


def _prep_kernel(x_ref, attr_ref, u_ref, emb_ref, tab_ref):
    xb = x_ref[...]
    vf = jnp.dot(xb, u_ref[...], preferred_element_type=jnp.float32)
    oh = (attr_ref[...] == lax.broadcasted_iota(
        jnp.int32, (NODE_BLOCK, NMAX_ATOMS), 1)).astype(jnp.float32)
    zb = jnp.dot(oh, emb_ref[...], preferred_element_type=jnp.float32)
    tab_ref[...] = jnp.concatenate(
        [jnp.zeros((NODE_BLOCK, N_SCALAR), jnp.float32), vf, zb], axis=1)


def _prep_nodes(x, attr2, U, embedding):
    nblk = N_NODES // NODE_BLOCK
    return pl.pallas_call(
        _prep_kernel,
        grid=(nblk,),
        in_specs=[
            pl.BlockSpec((NODE_BLOCK, 6), lambda i: (i, 0)),
            pl.BlockSpec((NODE_BLOCK, 1), lambda i: (i, 0)),
            pl.BlockSpec((6, 24), lambda i: (0, 0)),
            pl.BlockSpec((NMAX_ATOMS, EMBED_DIM), lambda i: (0, 0)),
        ],
        out_specs=pl.BlockSpec((NODE_BLOCK, 48), lambda i: (i, 0)),
        out_shape=jax.ShapeDtypeStruct((N_NODES, 48), jnp.float32),
    )(x, attr2, U, embedding)


def _node_kernel(tab_ref, p0_ref, p1_ref, p2_ref, sis_ref, sive_ref, r_ref,
                 p_ref, out_ref, *, final):
    tab = tab_ref[...]
    agg = jnp.concatenate([
        p0_ref[0] + p0_ref[1],
        p1_ref[0] + p1_ref[1],
        p2_ref[0] + p2_ref[1],
    ], axis=1) * INV_SQRT_DEG
    s_old = tab[:, :N_SCALAR]
    vflat_old = tab[:, N_SCALAR:N_SCALAR + 24]
    scal = s_old + agg[:, :N_SCALAR]
    gates = agg[:, N_SCALAR:24]
    s_new = jax.nn.silu(jnp.dot(scal, sis_ref[...],
                                preferred_element_type=jnp.float32))
    w = vflat_old + agg[:, 24:]
    vflat_new = (jnp.dot(jax.nn.sigmoid(gates), r_ref[...],
                         preferred_element_type=jnp.float32) *
                 jnp.dot(w, sive_ref[...], preferred_element_type=jnp.float32))
    if final:
        out_ref[...] = jnp.dot(vflat_new, p_ref[...],
                               preferred_element_type=jnp.float32)
    else:
        out_ref[...] = jnp.concatenate(
            [s_new, vflat_new, tab[:, N_SCALAR + 24:]], axis=1)


def _node_update(table, p0, p1, p2, SIs, SIvE, R, P, final):
    nblk = N_NODES // NODE_BLOCK
    out_w = 6 if final else 48
    return pl.pallas_call(
        functools.partial(_node_kernel, final=final),
        grid=(nblk,),
        in_specs=[
            pl.BlockSpec((NODE_BLOCK, 48), lambda i: (i, 0)),
            pl.BlockSpec((2, NODE_BLOCK, 16), lambda i: (0, i, 0)),
            pl.BlockSpec((2, NODE_BLOCK, 16), lambda i: (0, i, 0)),
            pl.BlockSpec((2, NODE_BLOCK, 16), lambda i: (0, i, 0)),
            pl.BlockSpec((N_SCALAR, N_SCALAR), lambda i: (0, 0)),
            pl.BlockSpec((24, 24), lambda i: (0, 0)),
            pl.BlockSpec((N_VEC, 24), lambda i: (0, 0)),
            pl.BlockSpec((24, 6), lambda i: (0, 0)),
        ],
        out_specs=pl.BlockSpec((NODE_BLOCK, out_w), lambda i: (i, 0)),
        out_shape=jax.ShapeDtypeStruct((N_NODES, out_w), jnp.float32),
    )(table, p0, p1, p2, SIs, SIvE, R, P)


def kernel(x, batch, node_attr, edge_index, edge_vec, projection_matrix, embedding,
           W_r1, b_r1, W_r2, b_r2, W_e, W_s, W_v1, W_v2, SI_s, SI_v):
    src = edge_index[0]
    dst = edge_index[1]
    Mu = _semi_unitary(projection_matrix)
    R = _expand_rep3(N_VEC)
    U = jnp.zeros((6, 3 * N_VEC), jnp.float32)
    for k in range(2):
        for d in range(3):
            U = U.at[3 * k + d, d::3].set(Mu[k, :])
    P = jnp.zeros((3 * N_VEC, 6), jnp.float32)
    for k in range(2):
        for d in range(3):
            P = P.at[d::3, 3 * k + d].set(Mu[k, :])
    table = _prep_nodes(x, node_attr.reshape(N_NODES, 1).astype(jnp.int32),
                        U, embedding)
    src2 = src.reshape(1, N_EDGES)
    dst2 = dst.reshape(1, N_EDGES)
    zeros16 = jnp.zeros((N_NODES, 16), jnp.float32)
    out = None
    for l in range(LAYERS):
        wcomb = jnp.zeros((56, 48), jnp.float32)
        wcomb = wcomb.at[:N_SCALAR, :RAD_OUT].set(W_e[l, :N_SCALAR, :])
        wcomb = wcomb.at[N_SCALAR:N_SCALAR + 24, :RAD_OUT].set(
            R.T @ W_e[l, N_SCALAR:N_SCALAR + N_VEC, :])
        wcomb = wcomb.at[N_SCALAR + 24:N_SCALAR + 32, :RAD_OUT].set(
            W_e[l, N_SCALAR + N_VEC:, :])
        wcomb = wcomb.at[N_SCALAR + 32:, RAD_OUT:].set(W_r1[l])
        wout = jnp.concatenate([W_s[l], W_v1[l] @ R, W_v2[l] @ R], axis=1)
        g = _sc_gather(table, src2)
        msg = _edge_stage(edge_vec, g, wcomb, b_r1[l][None, :], W_r2[l],
                          b_r2[l][None, :], wout)
        p0 = _sc_scatter(msg, dst2, zeros16, 0)
        p1 = _sc_scatter(msg, dst2, zeros16, 1)
        p2 = _sc_scatter(msg, dst2, zeros16, 2)
        SIvE = jnp.zeros((24, 24), jnp.float32)
        for d in range(3):
            SIvE = SIvE.at[d::3, d::3].set(SI_v[l])
        final = l == LAYERS - 1
        nxt = _node_update(table, p0, p1, p2, SI_s[l], SIvE, R, P, final)
        if final:
            out = nxt
        else:
            table = nxt
    return out


# speedup vs baseline: 104.9285x; 1.0006x over previous
import functools

import jax
import jax.numpy as jnp
import numpy as np
from jax import lax
from jax.experimental import pallas as pl
from jax.experimental.pallas import tpu as pltpu
from jax.experimental.pallas import tpu_sc as plsc

N_NODES = 100000
N_EDGES = 1600000
N_SCALAR = 16
N_VEC = 8
EMBED_DIM = 8
N_BASIS = 8
RAD_H = 16
RAD_OUT = 32
FEAT_DIM = N_SCALAR + N_VEC + EMBED_DIM
LAYERS = 2
MAX_RADIUS = 5.0
INV_SQRT_DEG = 1.0 / np.sqrt(16.0)

EDGE_BLOCK = 2000


def _semi_unitary(M):
    I = jnp.eye(M.shape[-2], dtype=M.dtype)
    for _ in range(10):
        M = M - 0.5 * (M @ M.T - I) @ M
    return M


def _expand_rep3(n):
    R = np.zeros((n, 3 * n), dtype=np.float32)
    for c in range(n):
        for d in range(3):
            R[c, 3 * c + d] = 1.0
    return jnp.asarray(R)


GATHER_WIN = 128


def _sc_mesh():
    return plsc.VectorSubcoreMesh(core_axis_name="c", subcore_axis_name="s")


GBATCH = 5


def _sc_gather(table, idx, width):
    n_win = N_EDGES // GATHER_WIN
    idx2 = idx.reshape(1, N_EDGES)

    @functools.partial(
        pl.kernel,
        out_type=jax.ShapeDtypeStruct((N_EDGES, width), jnp.float32),
        mesh=_sc_mesh(),
        compiler_params=pltpu.CompilerParams(use_tc_tiling_on_sc=False),
    )
    def k(table_hbm, idx_hbm, out_hbm):
        def body(i_vmem, o_vmem):
            pltpu.sync_copy(table_hbm.at[i_vmem.at[0]], o_vmem)

        pltpu.emit_pipeline(
            body,
            grid=(n_win,),
            in_specs=[pl.BlockSpec((1, GATHER_WIN), index_map=lambda i: (0, i))],
            out_specs=[pl.BlockSpec((GATHER_WIN, width),
                                    index_map=lambda i: (i, 0))],
            core_axis_name=("c", "s"),
            dimension_semantics=(pltpu.PARALLEL,),
        )(idx_hbm, out_hbm)

    return k(table, idx2)


SUB_ROWS = N_NODES // 16


def _sc_scatter3(msg, idx2, zeros):
    n_win = N_EDGES // GATHER_WIN

    @functools.partial(
        pl.kernel,
        out_type=jax.ShapeDtypeStruct((3, 2, N_NODES, 16), jnp.float32),
        mesh=_sc_mesh(),
        scratch_types=[pltpu.VMEM_SHARED((N_NODES, 16), jnp.float32)],
        compiler_params=pltpu.CompilerParams(use_tc_tiling_on_sc=False),
    )
    def k(msg_hbm, idx_hbm, zeros_hbm, out_hbm, acc):
        cid = lax.axis_index("c")
        sid = lax.axis_index("s")
        sl = pl.ds(sid * SUB_ROWS, SUB_ROWS)

        def body(x_vmem, i_vmem):
            pltpu.sync_copy(x_vmem, acc.at[i_vmem.at[0]], add=True)

        for grp in range(3):
            pltpu.sync_copy(zeros_hbm.at[sl], acc.at[sl])
            plsc.subcore_barrier()
            pltpu.emit_pipeline(
                body,
                grid=(n_win,),
                in_specs=[
                    pl.BlockSpec((GATHER_WIN, 16),
                                 index_map=functools.partial(
                                     lambda g, i: (i, g), grp)),
                    pl.BlockSpec((1, GATHER_WIN), index_map=lambda i: (0, i)),
                ],
                out_specs=[],
                core_axis_name=("c", "s"),
                dimension_semantics=(pltpu.PARALLEL,),
            )(msg_hbm, idx_hbm)
            plsc.subcore_barrier()
            pltpu.sync_copy(acc.at[sl], out_hbm.at[grp].at[cid].at[sl])

    return k(msg, idx2, zeros)


def _edge_kernel(ev_ref, g_ref, wcomb_ref, br1_ref, wr2_ref, br2_ref, wout_ref,
                 msg_ref, *, has_s):
    ev = ev_ref[...]
    g = g_ref[...]
    elen = jnp.sqrt(jnp.sum(ev * ev, axis=1, keepdims=True) + 1e-12)
    edir = ev / elen
    step = MAX_RADIUS / (N_BASIS - 1)
    values = lax.broadcasted_iota(jnp.int32, (1, N_BASIS), 1).astype(jnp.float32) * step
    diff = (elen - values) / step
    basis = jnp.exp(-diff * diff) * 0.5
    edir_t = jnp.concatenate([edir] * N_VEC, axis=1)
    off = N_SCALAR if has_s else 0
    vflat = g[:, off:off + 3 * N_VEC]
    parts = ([g[:, :N_SCALAR]] if has_s else []) + [
        vflat * edir_t, g[:, off + 3 * N_VEC:], basis]
    xin = jnp.concatenate(parts, axis=1)
    hr = jnp.dot(xin, wcomb_ref[...], preferred_element_type=jnp.float32)
    h_pre = hr[:, :RAD_OUT]
    r_pre = hr[:, RAD_OUT:] + br1_ref[...]
    radial = jnp.dot(jax.nn.silu(r_pre), wr2_ref[...],
                     preferred_element_type=jnp.float32) + br2_ref[...]
    h = jax.nn.silu(h_pre) * radial
    o = jnp.dot(h, wout_ref[...], preferred_element_type=jnp.float32)
    msg_v = o[:, 24:48] * edir_t + o[:, 48:72] * vflat
    msg_ref[...] = jnp.concatenate([o[:, :24], msg_v], axis=1)


def _edge_stage(ev, g, wcomb, br1, wr2, br2, wout, has_s):
    nblk = N_EDGES // EDGE_BLOCK
    gw, cw = (48, 56) if has_s else (32, 40)
    return pl.pallas_call(
        functools.partial(_edge_kernel, has_s=has_s),
        grid=(nblk,),
        in_specs=[
            pl.BlockSpec((EDGE_BLOCK, 3), lambda i: (i, 0)),
            pl.BlockSpec((EDGE_BLOCK, gw), lambda i: (i, 0)),
            pl.BlockSpec((cw, 48), lambda i: (0, 0)),
            pl.BlockSpec((1, RAD_H), lambda i: (0, 0)),
            pl.BlockSpec((RAD_H, RAD_OUT), lambda i: (0, 0)),
            pl.BlockSpec((1, RAD_OUT), lambda i: (0, 0)),
            pl.BlockSpec((RAD_OUT, 72), lambda i: (0, 0)),
        ],
        out_specs=pl.BlockSpec((EDGE_BLOCK, 48), lambda i: (i, 0)),
        out_shape=jax.ShapeDtypeStruct((N_EDGES, 48), jnp.float32),
    )(ev, g, wcomb, br1, wr2, br2, wout)


NODE_BLOCK = 2000
NMAX_ATOMS = 20


def _prep_kernel(x_ref, attr_ref, u_ref, emb_ref, tab_ref):
    xb = x_ref[...]
    vf = jnp.dot(xb, u_ref[...], preferred_element_type=jnp.float32)
    oh = (attr_ref[...] == lax.broadcasted_iota(
        jnp.int32, (NODE_BLOCK, NMAX_ATOMS), 1)).astype(jnp.float32)
    zb = jnp.dot(oh, emb_ref[...], preferred_element_type=jnp.float32)
    tab_ref[...] = jnp.concatenate([vf, zb], axis=1)


def _prep_nodes(x, attr2, U, embedding):
    nblk = N_NODES // NODE_BLOCK
    return pl.pallas_call(
        _prep_kernel,
        grid=(nblk,),
        in_specs=[
            pl.BlockSpec((NODE_BLOCK, 6), lambda i: (i, 0)),
            pl.BlockSpec((NODE_BLOCK, 1), lambda i: (i, 0)),
            pl.BlockSpec((6, 24), lambda i: (0, 0)),
            pl.BlockSpec((NMAX_ATOMS, EMBED_DIM), lambda i: (0, 0)),
        ],
        out_specs=pl.BlockSpec((NODE_BLOCK, 32), lambda i: (i, 0)),
        out_shape=jax.ShapeDtypeStruct((N_NODES, 32), jnp.float32),
    )(x, attr2, U, embedding)


def _node_kernel(tab_ref, p_ref, sis_ref, sive_ref, r_ref,
                 pmat_ref, out_ref, *, final, has_s):
    tab = tab_ref[...]
    agg = jnp.concatenate([
        p_ref[0, 0] + p_ref[0, 1],
        p_ref[1, 0] + p_ref[1, 1],
        p_ref[2, 0] + p_ref[2, 1],
    ], axis=1) * INV_SQRT_DEG
    off = N_SCALAR if has_s else 0
    vflat_old = tab[:, off:off + 24]
    scal = tab[:, :N_SCALAR] + agg[:, :N_SCALAR] if has_s else agg[:, :N_SCALAR]
    gates = agg[:, N_SCALAR:24]
    s_new = jax.nn.silu(jnp.dot(scal, sis_ref[...],
                                preferred_element_type=jnp.float32))
    w = vflat_old + agg[:, 24:]
    vflat_new = (jnp.dot(jax.nn.sigmoid(gates), r_ref[...],
                         preferred_element_type=jnp.float32) *
                 jnp.dot(w, sive_ref[...], preferred_element_type=jnp.float32))
    if final:
        out_ref[...] = jnp.dot(vflat_new, pmat_ref[...],
                               preferred_element_type=jnp.float32)
    else:
        out_ref[...] = jnp.concatenate(
            [s_new, vflat_new, tab[:, off + 24:]], axis=1)


def _node_update(table, parts, SIs, SIvE, R, P, final, has_s):
    nblk = N_NODES // NODE_BLOCK
    out_w = 6 if final else 48
    in_w = 48 if has_s else 32
    return pl.pallas_call(
        functools.partial(_node_kernel, final=final, has_s=has_s),
        grid=(nblk,),
        in_specs=[
            pl.BlockSpec((NODE_BLOCK, in_w), lambda i: (i, 0)),
            pl.BlockSpec((3, 2, NODE_BLOCK, 16), lambda i: (0, 0, i, 0)),
            pl.BlockSpec((N_SCALAR, N_SCALAR), lambda i: (0, 0)),
            pl.BlockSpec((24, 24), lambda i: (0, 0)),
            pl.BlockSpec((N_VEC, 24), lambda i: (0, 0)),
            pl.BlockSpec((24, 6), lambda i: (0, 0)),
        ],
        out_specs=pl.BlockSpec((NODE_BLOCK, out_w), lambda i: (i, 0)),
        out_shape=jax.ShapeDtypeStruct((N_NODES, out_w), jnp.float32),
    )(table, parts, SIs, SIvE, R, P)


def kernel(x, batch, node_attr, edge_index, edge_vec, projection_matrix, embedding,
           W_r1, b_r1, W_r2, b_r2, W_e, W_s, W_v1, W_v2, SI_s, SI_v):
    src = edge_index[0]
    dst = edge_index[1]
    Mu = _semi_unitary(projection_matrix)
    R = _expand_rep3(N_VEC)
    U = jnp.zeros((6, 3 * N_VEC), jnp.float32)
    for k in range(2):
        for d in range(3):
            U = U.at[3 * k + d, d::3].set(Mu[k, :])
    P = jnp.zeros((3 * N_VEC, 6), jnp.float32)
    for k in range(2):
        for d in range(3):
            P = P.at[d::3, 3 * k + d].set(Mu[k, :])
    table = _prep_nodes(x, node_attr.reshape(N_NODES, 1).astype(jnp.int32),
                        U, embedding)
    src2 = src.reshape(1, N_EDGES)
    dst2 = dst.reshape(1, N_EDGES)
    zeros16 = jnp.zeros((N_NODES, 16), jnp.float32)
    out = None
    for l in range(LAYERS):
        has_s = l > 0
        off = N_SCALAR if has_s else 0
        wcomb = jnp.zeros((off + 40, 48), jnp.float32)
        if has_s:
            wcomb = wcomb.at[:N_SCALAR, :RAD_OUT].set(W_e[l, :N_SCALAR, :])
        wcomb = wcomb.at[off:off + 24, :RAD_OUT].set(
            R.T @ W_e[l, N_SCALAR:N_SCALAR + N_VEC, :])
        wcomb = wcomb.at[off + 24:off + 32, :RAD_OUT].set(
            W_e[l, N_SCALAR + N_VEC:, :])
        wcomb = wcomb.at[off + 32:, RAD_OUT:].set(W_r1[l])
        wout = jnp.concatenate([W_s[l], W_v1[l] @ R, W_v2[l] @ R], axis=1)
        g = _sc_gather(table, src2, 48 if has_s else 32)
        msg = _edge_stage(edge_vec, g, wcomb, b_r1[l][None, :], W_r2[l],
                          b_r2[l][None, :], wout, has_s)
        parts = _sc_scatter3(msg, dst2, zeros16)
        SIvE = jnp.zeros((24, 24), jnp.float32)
        for d in range(3):
            SIvE = SIvE.at[d::3, d::3].set(SI_v[l])
        final = l == LAYERS - 1
        nxt = _node_update(table, parts, SI_s[l], SIvE, R, P, final, has_s)
        if final:
            out = nxt
        else:
            table = nxt
    return out
